# Initial kernel scaffold; baseline (speedup 1.0000x reference)
import jax
import jax.numpy as jnp
from jax.experimental import pallas as pl


def kernel(x, super_points, W_down, b_down, W_up, b_up):
    raise NotImplementedError("write your pallas kernel here")


# speedup vs baseline: 6.5523x; 6.5523x over previous
import jax
import jax.numpy as jnp
from jax.experimental import pallas as pl

_B, _C, _H, _W = 4, 192, 14, 14
_N = _H * _W
_T = _B * _N
_D = 32
_K = 1024


def _body(x_ref, sp_ref, spt_ref, wd_ref, bd_ref, wu_ref, bu_ref,
          q_ref, idx_ref, s_ref):
    wd = wd_ref[...]
    ys = [jnp.dot(wd, x_ref[b], preferred_element_type=jnp.float32)
          for b in range(_B)]
    y = jnp.concatenate(ys, axis=1) + bd_ref[...]

    norms = jnp.sqrt(jnp.sum(y * y, axis=0, keepdims=True))
    xn = y / norms

    sp = sp_ref[...]
    dots_hi = jnp.dot(sp, xn, precision=jax.lax.Precision.HIGHEST,
                      preferred_element_type=jnp.float32)
    spn2 = jnp.sum(sp * sp, axis=1, keepdims=True)
    score = spn2 - 2.0 * dots_hi
    smin = jnp.min(score, axis=0, keepdims=True)
    kiota = jax.lax.broadcasted_iota(jnp.int32, (_K, _T), 0)
    idx = jnp.min(jnp.where(score == smin, kiota, jnp.int32(1 << 30)),
                  axis=0, keepdims=True)
    idx_ref[...] = idx

    dots = jnp.dot(sp, xn, preferred_element_type=jnp.float32)
    dmax = jnp.max(dots, axis=0, keepdims=True)
    logits = dots * 100.0
    cmax = dmax * 100.0
    ex = jnp.exp(logits - cmax)
    sums = jnp.sum(ex, axis=0, keepdims=True)
    probs = ex / sums
    log_probs = (logits - cmax) - jnp.log(sums)
    sample_entropy = -jnp.sum(probs * log_probs, axis=0, keepdims=True)
    em = jnp.sum(sample_entropy, axis=1, keepdims=True) / float(_T)
    ap = jnp.sum(probs, axis=1, keepdims=True) / float(_T)
    me = -jnp.sum(ap * jnp.log(ap + 1e-15), axis=0, keepdims=True)

    oh = (kiota == idx).astype(jnp.float32)
    quant = jnp.dot(spt_ref[...], oh, precision=jax.lax.Precision.HIGHEST,
                    preferred_element_type=jnp.float32)

    st = xn + (quant - xn)
    diff = xn - st
    cl = jnp.sum(jnp.sum(diff * diff, axis=0, keepdims=True),
                 axis=1, keepdims=True) / float(_D * _T)

    s_ref[...] = jnp.concatenate([em, me, em - me, cl], axis=1)

    wu = wu_ref[...]
    bu = bu_ref[...]
    for b in range(_B):
        q_ref[b] = jnp.dot(wu, st[:, b * _N:(b + 1) * _N],
                           preferred_element_type=jnp.float32) + bu


def kernel(x, super_points, W_down, b_down, W_up, b_up):
    x3 = x.reshape(_B, _C, _N)
    q3, idx, s = pl.pallas_call(
        _body,
        out_shape=(
            jax.ShapeDtypeStruct((_B, _C, _N), jnp.float32),
            jax.ShapeDtypeStruct((1, _T), jnp.int32),
            jax.ShapeDtypeStruct((1, 4), jnp.float32),
        ),
    )(x3, super_points, super_points.T, W_down,
      b_down.reshape(_D, 1), W_up, b_up.reshape(_C, 1))
    q = q3.reshape(_B, _C, _H, _W)
    idx_flat = idx.reshape(-1)
    return (q, idx_flat, s[0, 0], s[0, 1], s[0, 2], s[0, 3])


# speedup vs baseline: 7.6392x; 1.1659x over previous
import jax
import jax.numpy as jnp
from jax.experimental import pallas as pl

_B, _C, _H, _W = 4, 192, 14, 14
_N = _H * _W
_T = _B * _N
_D = 32
_K = 1024


def _body(x_ref, sp_ref, wd_ref, bd_ref, wu_ref, bu_ref,
          q_ref, idx_ref, s_ref):
    xall = jnp.concatenate([x_ref[b] for b in range(_B)], axis=1)
    y = jnp.dot(wd_ref[...], xall,
                preferred_element_type=jnp.float32)
    y = y + bd_ref[...].reshape(_D, 1)

    norms = jnp.sqrt(jnp.sum(y * y, axis=0, keepdims=True))
    xn = y / norms

    sp = sp_ref[...]
    dots = jnp.dot(sp, xn, precision=jax.lax.Precision.HIGHEST,
                   preferred_element_type=jnp.float32)
    spn2 = jnp.sum(sp * sp, axis=1, keepdims=True)
    score = spn2 - 2.0 * dots
    smin = jnp.min(score, axis=0, keepdims=True)
    kiota = jax.lax.broadcasted_iota(jnp.int32, (_K, _T), 0)
    idx = jnp.min(jnp.where(score == smin, kiota, jnp.int32(1 << 30)),
                  axis=0, keepdims=True)
    idx_ref[...] = idx

    logits = dots * 100.0
    cmax = jnp.max(dots, axis=0, keepdims=True) * 100.0
    ex = jnp.exp(logits - cmax)
    sums = jnp.sum(ex, axis=0, keepdims=True)
    probs = ex / sums
    log_probs = (logits - cmax) - jnp.log(sums)
    sample_entropy = -jnp.sum(probs * log_probs, axis=0, keepdims=True)
    em = jnp.sum(sample_entropy, axis=1, keepdims=True) / float(_T)
    ap = jnp.sum(probs, axis=1, keepdims=True) / float(_T)
    me = -jnp.sum(ap * jnp.log(ap + 1e-15), axis=0, keepdims=True)

    oh = (kiota == idx).astype(jnp.float32)
    quant = jax.lax.dot_general(
        sp, oh, (((0,), (0,)), ((), ())),
        precision=jax.lax.Precision.HIGHEST,
        preferred_element_type=jnp.float32)

    st = xn + (quant - xn)
    diff = xn - st
    cl = jnp.sum(jnp.sum(diff * diff, axis=0, keepdims=True),
                 axis=1, keepdims=True) / float(_D * _T)

    s_ref[...] = jnp.concatenate([em, me, em - me, cl], axis=1)

    q = jnp.dot(wu_ref[...], st,
                preferred_element_type=jnp.float32)
    q = q + bu_ref[...].reshape(_C, 1)
    for b in range(_B):
        q_ref[b] = q[:, b * _N:(b + 1) * _N]


def kernel(x, super_points, W_down, b_down, W_up, b_up):
    x3 = x.reshape(_B, _C, _N)
    q3, idx, s = pl.pallas_call(
        _body,
        out_shape=(
            jax.ShapeDtypeStruct((_B, _C, _N), jnp.float32),
            jax.ShapeDtypeStruct((1, _T), jnp.int32),
            jax.ShapeDtypeStruct((1, 4), jnp.float32),
        ),
    )(x3, super_points, W_down, b_down, W_up, b_up)
    q = q3.reshape(_B, _C, _H, _W)
    idx_flat = idx.reshape(-1)
    return (q, idx_flat, s[0, 0], s[0, 1], s[0, 2], s[0, 3])


# speedup vs baseline: 8.1553x; 1.0675x over previous
import jax
import jax.numpy as jnp
from jax.experimental import pallas as pl

_B, _C, _H, _W = 4, 192, 14, 14
_N = _H * _W
_T = _B * _N
_D = 32
_K = 1024


def _body(x_ref, sp_ref, wd_ref, bd_ref, wu_ref, bu_ref,
          q_ref, idx_ref, em_ref, me_ref, el_ref, cl_ref):
    xall = jnp.concatenate([x_ref[b] for b in range(_B)], axis=1)
    y = jnp.dot(wd_ref[...], xall,
                preferred_element_type=jnp.float32)
    y = y + bd_ref[...].reshape(_D, 1)

    norms = jnp.sqrt(jnp.sum(y * y, axis=0, keepdims=True))
    xn = y / norms

    sp = sp_ref[...]
    dots = jnp.dot(sp, xn, precision=jax.lax.Precision.HIGHEST,
                   preferred_element_type=jnp.float32)
    spn2 = jnp.sum(sp * sp, axis=1, keepdims=True)
    score = spn2 - 2.0 * dots
    smin = jnp.min(score, axis=0, keepdims=True)
    kiota = jax.lax.broadcasted_iota(jnp.int32, (_K, _T), 0)
    idx = jnp.min(jnp.where(score == smin, kiota, jnp.int32(1 << 30)),
                  axis=0, keepdims=True)
    idx_ref[...] = idx

    logits = dots * 100.0
    cmax = jnp.max(dots, axis=0, keepdims=True) * 100.0
    ex = jnp.exp(logits - cmax)
    sums = jnp.sum(ex, axis=0, keepdims=True)
    probs = ex / sums
    log_probs = (logits - cmax) - jnp.log(sums)
    sample_entropy = -jnp.sum(probs * log_probs, axis=0, keepdims=True)
    em = jnp.sum(sample_entropy, axis=1, keepdims=True) / float(_T)
    ap = jnp.sum(probs, axis=1, keepdims=True) / float(_T)
    me = -jnp.sum(ap * jnp.log(ap + 1e-15), axis=0, keepdims=True)

    oh = (kiota == idx).astype(jnp.float32)
    quant = jax.lax.dot_general(
        sp, oh, (((0,), (0,)), ((), ())),
        precision=jax.lax.Precision.HIGHEST,
        preferred_element_type=jnp.float32)

    st = xn + (quant - xn)
    diff = xn - st
    cl = jnp.sum(jnp.sum(diff * diff, axis=0, keepdims=True),
                 axis=1, keepdims=True) / float(_D * _T)

    em_ref[...] = em
    me_ref[...] = me
    el_ref[...] = em - me
    cl_ref[...] = cl

    q = jnp.dot(wu_ref[...], st,
                preferred_element_type=jnp.float32)
    q = q + bu_ref[...].reshape(_C, 1)
    for b in range(_B):
        q_ref[b] = q[:, b * _N:(b + 1) * _N]


def kernel(x, super_points, W_down, b_down, W_up, b_up):
    x3 = x.reshape(_B, _C, _N)
    q3, idx, em, me, el, cl = pl.pallas_call(
        _body,
        out_shape=(
            jax.ShapeDtypeStruct((_B, _C, _N), jnp.float32),
            jax.ShapeDtypeStruct((1, _T), jnp.int32),
            jax.ShapeDtypeStruct((1, 1), jnp.float32),
            jax.ShapeDtypeStruct((1, 1), jnp.float32),
            jax.ShapeDtypeStruct((1, 1), jnp.float32),
            jax.ShapeDtypeStruct((1, 1), jnp.float32),
        ),
    )(x3, super_points, W_down, b_down, W_up, b_up)
    q = q3.reshape(_B, _C, _H, _W)
    idx_flat = idx.reshape(-1)
    return (q, idx_flat, em.reshape(()), me.reshape(()),
            el.reshape(()), cl.reshape(()))


# speedup vs baseline: 8.8290x; 1.0826x over previous
import jax
import jax.numpy as jnp
from jax.experimental import pallas as pl

_B, _C, _H, _W = 4, 192, 14, 14
_N = _H * _W
_T = _B * _N
_D = 32
_K = 1024


def _body(x_ref, sp_ref, wd_ref, bd_ref, wu_ref, bu_ref,
          q_ref, idx_ref, em_ref, me_ref, el_ref, cl_ref):
    xall = jnp.concatenate([x_ref[b] for b in range(_B)], axis=1)
    y = jnp.dot(wd_ref[...], xall,
                preferred_element_type=jnp.float32)
    y = y + bd_ref[...].reshape(_D, 1)

    norms = jnp.sqrt(jnp.sum(y * y, axis=0, keepdims=True))
    xn = y / norms

    sp = sp_ref[...]
    dots = jnp.dot(sp, xn, precision=jax.lax.Precision.HIGHEST,
                   preferred_element_type=jnp.float32)
    spn2 = jnp.sum(sp * sp, axis=1, keepdims=True)
    score = spn2 - 2.0 * dots
    smin = jnp.min(score, axis=0, keepdims=True)
    kiota = jax.lax.broadcasted_iota(jnp.int32, (_K, _T), 0)
    idx = jnp.min(jnp.where(score == smin, kiota, jnp.int32(1 << 30)),
                  axis=0, keepdims=True)
    idx_ref[...] = idx

    dmax = jnp.max(dots, axis=0, keepdims=True)
    z = (dots - dmax) * 100.0
    ex = jnp.exp(z)
    sums = jnp.sum(ex, axis=0, keepdims=True)
    probs = ex * (1.0 / sums)
    log_probs = z - jnp.log(sums)
    sample_entropy = -jnp.sum(probs * log_probs, axis=0, keepdims=True)
    em = jnp.sum(sample_entropy, axis=1, keepdims=True) / float(_T)
    ap = jnp.sum(probs, axis=1, keepdims=True) / float(_T)
    me = -jnp.sum(ap * jnp.log(ap + 1e-15), axis=0, keepdims=True)

    oh = (kiota == idx).astype(jnp.float32)
    quant = jax.lax.dot_general(
        sp, oh, (((0,), (0,)), ((), ())),
        preferred_element_type=jnp.float32)

    st = xn + (quant - xn)
    diff = xn - st
    cl = jnp.sum(jnp.sum(diff * diff, axis=0, keepdims=True),
                 axis=1, keepdims=True) / float(_D * _T)

    em_ref[...] = em
    me_ref[...] = me
    el_ref[...] = em - me
    cl_ref[...] = cl

    q = jnp.dot(wu_ref[...], st,
                preferred_element_type=jnp.float32)
    q = q + bu_ref[...].reshape(_C, 1)
    for b in range(_B):
        q_ref[b] = q[:, b * _N:(b + 1) * _N]


def kernel(x, super_points, W_down, b_down, W_up, b_up):
    x3 = x.reshape(_B, _C, _N)
    q3, idx, em, me, el, cl = pl.pallas_call(
        _body,
        out_shape=(
            jax.ShapeDtypeStruct((_B, _C, _N), jnp.float32),
            jax.ShapeDtypeStruct((1, _T), jnp.int32),
            jax.ShapeDtypeStruct((1, 1), jnp.float32),
            jax.ShapeDtypeStruct((1, 1), jnp.float32),
            jax.ShapeDtypeStruct((1, 1), jnp.float32),
            jax.ShapeDtypeStruct((1, 1), jnp.float32),
        ),
    )(x3, super_points, W_down, b_down, W_up, b_up)
    q = q3.reshape(_B, _C, _H, _W)
    idx_flat = idx.reshape(-1)
    return (q, idx_flat, em.reshape(()), me.reshape(()),
            el.reshape(()), cl.reshape(()))


# speedup vs baseline: 9.0028x; 1.0197x over previous
import jax
import jax.numpy as jnp
from jax.experimental import pallas as pl

_B, _C, _H, _W = 4, 192, 14, 14
_N = _H * _W
_T = _B * _N
_D = 32
_K = 1024


def _body(x_ref, sp_ref, wd_ref, bd_ref, wu_ref, bu_ref,
          q_ref, idx_ref, em_ref, me_ref, el_ref, cl_ref):
    xall = jnp.concatenate([x_ref[b] for b in range(_B)], axis=1)
    y = jnp.dot(wd_ref[...], xall,
                preferred_element_type=jnp.float32)
    y = y + bd_ref[...].reshape(_D, 1)

    norms = jnp.sqrt(jnp.sum(y * y, axis=0, keepdims=True))
    xn = y / norms

    sp = sp_ref[...]
    dots = jnp.dot(sp, xn, precision=jax.lax.Precision.HIGHEST,
                   preferred_element_type=jnp.float32)
    dmax = jnp.max(dots, axis=0, keepdims=True)
    kiota = jax.lax.broadcasted_iota(jnp.int32, (_K, _T), 0)
    idx = jnp.min(jnp.where(dots == dmax, kiota, jnp.int32(1 << 30)),
                  axis=0, keepdims=True)
    idx_ref[...] = idx

    z = (dots - dmax) * 100.0
    ex = jnp.exp(z)
    sums = jnp.sum(ex, axis=0, keepdims=True)
    inv = 1.0 / sums
    sez = jnp.sum(ex * z, axis=0, keepdims=True)
    sample_entropy = jnp.log(sums) - inv * sez
    em = jnp.sum(sample_entropy, axis=1, keepdims=True) / float(_T)
    ap = jnp.sum(ex * inv, axis=1, keepdims=True) / float(_T)
    me = -jnp.sum(ap * jnp.log(ap + 1e-15), axis=0, keepdims=True)

    oh = (kiota == idx).astype(jnp.float32)
    quant = jax.lax.dot_general(
        sp, oh, (((0,), (0,)), ((), ())),
        preferred_element_type=jnp.float32)

    st = xn + (quant - xn)
    diff = xn - st
    cl = jnp.sum(jnp.sum(diff * diff, axis=0, keepdims=True),
                 axis=1, keepdims=True) / float(_D * _T)

    em_ref[...] = em
    me_ref[...] = me
    el_ref[...] = em - me
    cl_ref[...] = cl

    q = jnp.dot(wu_ref[...], st,
                preferred_element_type=jnp.float32)
    q = q + bu_ref[...].reshape(_C, 1)
    for b in range(_B):
        q_ref[b] = q[:, b * _N:(b + 1) * _N]


def kernel(x, super_points, W_down, b_down, W_up, b_up):
    x3 = x.reshape(_B, _C, _N)
    q3, idx, em, me, el, cl = pl.pallas_call(
        _body,
        out_shape=(
            jax.ShapeDtypeStruct((_B, _C, _N), jnp.float32),
            jax.ShapeDtypeStruct((1, _T), jnp.int32),
            jax.ShapeDtypeStruct((1, 1), jnp.float32),
            jax.ShapeDtypeStruct((1, 1), jnp.float32),
            jax.ShapeDtypeStruct((1, 1), jnp.float32),
            jax.ShapeDtypeStruct((1, 1), jnp.float32),
        ),
    )(x3, super_points, W_down, b_down, W_up, b_up)
    q = q3.reshape(_B, _C, _H, _W)
    idx_flat = idx.reshape(-1)
    return (q, idx_flat, em.reshape(()), me.reshape(()),
            el.reshape(()), cl.reshape(()))
